# baseline (speedup 1.0000x reference)
import jax
import jax.numpy as jnp
from jax.experimental import pallas as pl
from jax.experimental.pallas import tpu as pltpu

_N = 50000
_E = 800000
_RB = 2000
_NBLK = _N // _RB
_CE = 12500
_EC = _E // _CE


def _full(a):
    nd = a.ndim
    return pl.BlockSpec(a.shape, lambda i, _n=nd: (0,) * _n)


def _conv3(xs, Ws, b):
    y0 = xs[0] @ Ws[1] + xs[1] @ Ws[2] + b
    y1 = xs[0] @ Ws[0] + xs[1] @ Ws[1] + xs[2] @ Ws[2] + b
    y2 = xs[1] @ Ws[0] + xs[2] @ Ws[1] + b
    return [y0, y1, y2]


def _gated(y, wA, bA, wB, bB, wC, bC):
    P = _conv3(y, wA, bA)
    Q = _conv3(y, wB, bB)
    R = _conv3(y, wC, bC)
    return [jnp.maximum(P[t] * jax.nn.sigmoid(Q[t]) + R[t], 0.0) for t in range(3)]


def _k1(x_ref, winT, bin_, wrT, br,
        w10, w11, w12, b1, w20, w21, w22, b2, w30, w31, w32, b3,
        h1_ref, res_ref, ps_ref, pss_ref):
    xt = [x_ref[:, t, :] for t in range(3)]
    y = [xt[t] @ winT[:] + bin_[:] for t in range(3)]
    r = [y[t] @ wrT[:] + br[:] for t in range(3)]
    h = _gated(y, [w10[:], w11[:], w12[:]], b1[:],
               [w20[:], w21[:], w22[:]], b2[:],
               [w30[:], w31[:], w32[:]], b3[:])
    h48 = jnp.concatenate(h, axis=1)
    h1_ref[:] = h48
    res_ref[:] = jnp.concatenate(r, axis=1)
    ps_ref[:] = jnp.sum(h48, axis=0, keepdims=True)[None]
    pss_ref[:] = jnp.sum(h48 * h48, axis=0, keepdims=True)[None]


def _k2(h1_ref, sc_ref, sh_ref, wgT_ref, asw_ref, adw_ref,
        hw0_ref, hw1_ref, asd_ref):
    hn = h1_ref[:] * sc_ref[:] + sh_ref[:]
    hw = hn @ wgT_ref[:]
    hw0_ref[:] = hw[:, 0:48]
    hw1_ref[:] = hw[:, 48:96]
    a = []
    for hd in range(2):
        blk = hw[:, hd * 48:(hd + 1) * 48]
        a.append(jnp.sum(blk * asw_ref[hd:hd + 1, :], axis=1, keepdims=True))
    for hd in range(2):
        blk = hw[:, hd * 48:(hd + 1) * 48]
        a.append(jnp.sum(blk * adw_ref[hd:hd + 1, :], axis=1, keepdims=True))
    asd_ref[:] = jnp.concatenate(a, axis=1)


def _k3a(src_ref, dst_ref, asd_ref, ex0_ref, ex1_ref, den_ref):
    @pl.when(pl.program_id(0) == 0)
    def _init():
        den_ref[:] = jnp.zeros((_N, 2), jnp.float32)

    def body(i, carry):
        s = src_ref[0, 0, i]
        d = dst_ref[0, 0, i]
        e = asd_ref[pl.ds(s, 1), 0:2] + asd_ref[pl.ds(d, 1), 2:4]
        e = jnp.where(e >= 0, e, 0.2 * e)
        ex = jnp.exp(e)
        den_ref[pl.ds(d, 1), :] = den_ref[pl.ds(d, 1), :] + ex
        ex0_ref[0, 0, i] = ex[0, 0]
        ex1_ref[0, 0, i] = ex[0, 1]
        return carry

    jax.lax.fori_loop(0, _CE, body, 0)


def _k3c(src_ref, dst_ref, ex_ref, hw_ref, out_ref):
    @pl.when(pl.program_id(0) == 0)
    def _init():
        out_ref[:] = jnp.zeros((_N, 48), jnp.float32)

    def body(i, carry):
        s = src_ref[0, 0, i]
        d = dst_ref[0, 0, i]
        w = ex_ref[0, 0, i]
        out_ref[pl.ds(d, 1), :] = out_ref[pl.ds(d, 1), :] + w * hw_ref[pl.ds(s, 1), :]
        return carry

    jax.lax.fori_loop(0, _CE, body, 0)


def _k4a(o0_ref, o1_ref, den_ref, g_ref, ps_ref, pss_ref):
    den = den_ref[:]
    g = 0.5 * (o0_ref[:] / (den[:, 0:1] + 1e-16) + o1_ref[:] / (den[:, 1:2] + 1e-16))
    g_ref[:] = g
    ps_ref[:] = jnp.sum(g, axis=0, keepdims=True)[None]
    pss_ref[:] = jnp.sum(g * g, axis=0, keepdims=True)[None]


def _k4b(g_ref, sc_ref, sh_ref,
         w10, w11, w12, b1, w20, w21, w22, b2, w30, w31, w32, b3,
         h2_ref, ps_ref, pss_ref):
    y = g_ref[:] * sc_ref[:] + sh_ref[:]
    xt = [y[:, 16 * t:16 * (t + 1)] for t in range(3)]
    h = _gated(xt, [w10[:], w11[:], w12[:]], b1[:],
               [w20[:], w21[:], w22[:]], b2[:],
               [w30[:], w31[:], w32[:]], b3[:])
    h48 = jnp.concatenate(h, axis=1)
    h2_ref[:] = h48
    ps_ref[:] = jnp.sum(h48, axis=0, keepdims=True)[None]
    pss_ref[:] = jnp.sum(h48 * h48, axis=0, keepdims=True)[None]


def _k5(h2_ref, sc_ref, sh_ref, res_ref, wo1T, bo1, wo2T, bo2, o_ref):
    h = h2_ref[:] * sc_ref[:] + sh_ref[:] + res_ref[:]
    h = jnp.maximum(h, 0.0)
    outs = []
    for t in range(3):
        ht = h[:, 16 * t:16 * (t + 1)]
        o1 = jnp.maximum(ht @ wo1T[:] + bo1[:], 0.0)
        outs.append(o1 @ wo2T[:] + bo2[:])
    o_ref[:] = jnp.concatenate(outs, axis=1)


def kernel(X, params, edge_index):
    p = params
    f32 = jnp.float32
    idx = jnp.arange(48)
    perm = (idx % 16) * 3 + idx // 16
    rperm = jnp.concatenate([perm, perm + 48])

    xT = jnp.transpose(X, (0, 2, 1))

    winT = p['W_in'][:, :, 0].T
    bin_ = p['b_in'][None, :]
    wrT = p['Wr'][:, :, 0].T
    br = p['br'][None, :]

    def taps(g):
        return ([g['W1'][:, :, k].T for k in range(3)] + [g['b1'][None, :]]
                + [g['W2'][:, :, k].T for k in range(3)] + [g['b2'][None, :]]
                + [g['W3'][:, :, k].T for k in range(3)] + [g['b3'][None, :]])

    k1w = [winT, bin_, wrT, br] + taps(p['gc1'])

    nb = pl.BlockSpec((_RB, 48), lambda i: (i, 0))
    pb = pl.BlockSpec((1, 1, 48), lambda i: (i, 0, 0))
    h1, res, ps1, pss1 = pl.pallas_call(
        _k1,
        grid=(_NBLK,),
        in_specs=[pl.BlockSpec((_RB, 3, 4), lambda i: (i, 0, 0))] + [_full(a) for a in k1w],
        out_specs=[nb, nb, pb, pb],
        out_shape=[jax.ShapeDtypeStruct((_N, 48), f32),
                   jax.ShapeDtypeStruct((_N, 48), f32),
                   jax.ShapeDtypeStruct((_NBLK, 1, 48), f32),
                   jax.ShapeDtypeStruct((_NBLK, 1, 48), f32)],
    )(xT, *k1w)

    cnt1 = float(_N * 3)
    s1 = jnp.sum(ps1, axis=(0, 1)).reshape(3, 16).sum(0)
    q1 = jnp.sum(pss1, axis=(0, 1)).reshape(3, 16).sum(0)
    m1 = s1 / cnt1
    v1 = q1 / cnt1 - m1 * m1
    sc1c = p['bn_g'] / jnp.sqrt(v1 + 1e-5)
    sh1c = p['bn_b'] - m1 * sc1c
    sc1 = jnp.tile(sc1c, 3)[None, :]
    sh1 = jnp.tile(sh1c, 3)[None, :]

    wg = p['Wg'][rperm][:, perm]
    wgT = wg.T
    asw = p['att_src'][:, perm]
    adw = p['att_dst'][:, perm]

    hw0, hw1, asd = pl.pallas_call(
        _k2,
        grid=(_NBLK,),
        in_specs=[nb, _full(sc1), _full(sh1), _full(wgT), _full(asw), _full(adw)],
        out_specs=[nb, nb, pl.BlockSpec((_RB, 4), lambda i: (i, 0))],
        out_shape=[jax.ShapeDtypeStruct((_N, 48), f32),
                   jax.ShapeDtypeStruct((_N, 48), f32),
                   jax.ShapeDtypeStruct((_N, 4), f32)],
    )(h1, sc1, sh1, wgT, asw, adw)

    src3 = edge_index[0].reshape(_EC, 1, _CE)
    dst3 = edge_index[1].reshape(_EC, 1, _CE)
    sm = pl.BlockSpec((1, 1, _CE), lambda i: (i, 0, 0), memory_space=pltpu.SMEM)
    exs = pl.BlockSpec((1, 1, _CE), lambda i: (i, 0, 0), memory_space=pltpu.SMEM)
    n2 = pl.BlockSpec((_N, 2), lambda i: (0, 0))
    n48 = pl.BlockSpec((_N, 48), lambda i: (0, 0))

    ex0, ex1, den = pl.pallas_call(
        _k3a,
        grid=(_EC,),
        in_specs=[sm, sm, _full(asd)],
        out_specs=[exs, exs, n2],
        out_shape=[jax.ShapeDtypeStruct((_EC, 1, _CE), f32),
                   jax.ShapeDtypeStruct((_EC, 1, _CE), f32),
                   jax.ShapeDtypeStruct((_N, 2), f32)],
    )(src3, dst3, asd)

    out0 = pl.pallas_call(
        _k3c,
        grid=(_EC,),
        in_specs=[sm, sm, exs, _full(hw0)],
        out_specs=n48,
        out_shape=jax.ShapeDtypeStruct((_N, 48), f32),
    )(src3, dst3, ex0, hw0)

    out1 = pl.pallas_call(
        _k3c,
        grid=(_EC,),
        in_specs=[sm, sm, exs, _full(hw1)],
        out_specs=n48,
        out_shape=jax.ShapeDtypeStruct((_N, 48), f32),
    )(src3, dst3, ex1, hw1)

    gat, ps4, pss4 = pl.pallas_call(
        _k4a,
        grid=(_NBLK,),
        in_specs=[nb, nb, pl.BlockSpec((_RB, 2), lambda i: (i, 0))],
        out_specs=[nb, pb, pb],
        out_shape=[jax.ShapeDtypeStruct((_N, 48), f32),
                   jax.ShapeDtypeStruct((_NBLK, 1, 48), f32),
                   jax.ShapeDtypeStruct((_NBLK, 1, 48), f32)],
    )(out0, out1, den)

    bg48 = p['bg'][perm]
    g1p = p['bn1_g'][perm]
    b1p = p['bn1_b'][perm]
    sg = jnp.sum(ps4, axis=(0, 1))
    qg = jnp.sum(pss4, axis=(0, 1))
    mg = sg / float(_N)
    vg = qg / float(_N) - mg * mg
    sc2f = g1p / jnp.sqrt(vg + 1e-5)
    sh2f = b1p + sc2f * (bg48 - (mg + bg48))
    sc2 = sc2f[None, :]
    sh2 = sh2f[None, :]

    k4w = taps(p['gc2'])
    h2, ps5, pss5 = pl.pallas_call(
        _k4b,
        grid=(_NBLK,),
        in_specs=[nb, _full(sc2), _full(sh2)] + [_full(a) for a in k4w],
        out_specs=[nb, pb, pb],
        out_shape=[jax.ShapeDtypeStruct((_N, 48), f32),
                   jax.ShapeDtypeStruct((_NBLK, 1, 48), f32),
                   jax.ShapeDtypeStruct((_NBLK, 1, 48), f32)],
    )(gat, sc2, sh2, *k4w)

    s5 = jnp.sum(ps5, axis=(0, 1)).reshape(3, 16).sum(0)
    q5 = jnp.sum(pss5, axis=(0, 1)).reshape(3, 16).sum(0)
    m5 = s5 / cnt1
    v5 = q5 / cnt1 - m5 * m5
    sc3c = p['bn2_g'] / jnp.sqrt(v5 + 1e-5)
    sh3c = p['bn2_b'] - m5 * sc3c
    sc3 = jnp.tile(sc3c, 3)[None, :]
    sh3 = jnp.tile(sh3c, 3)[None, :]

    wo1T = p['Wo1'][:, :, 0].T
    bo1 = p['bo1'][None, :]
    wo2T = p['Wo2'][:, :, 0].T
    bo2 = p['bo2'][None, :]

    o = pl.pallas_call(
        _k5,
        grid=(_NBLK,),
        in_specs=[nb, _full(sc3), _full(sh3), nb, _full(wo1T), _full(bo1),
                  _full(wo2T), _full(bo2)],
        out_specs=pl.BlockSpec((_RB, 3), lambda i: (i, 0)),
        out_shape=jax.ShapeDtypeStruct((_N, 3), f32),
    )(h2, sc3, sh3, res, wo1T, bo1, wo2T, bo2)

    return o.reshape(_N, 1, 3)
